# Initial kernel scaffold; baseline (speedup 1.0000x reference)
import jax
import jax.numpy as jnp
from jax.experimental import pallas as pl


def kernel(x, edge_index, W1, b1, W3, b3, W2, b2):
    raise NotImplementedError("write your pallas kernel here")


# speedup vs baseline: 12.9288x; 12.9288x over previous
import functools

import jax
import jax.numpy as jnp
from jax import lax
from jax.experimental import pallas as pl
from jax.experimental.pallas import tpu as pltpu
from jax.experimental.pallas import tpu_sc as plsc

N = 10000
E = 320000
D = 64
DIN = 128
DOUT = 128
NC, NS = 2, 16
NW = NC * NS
EPW = E // NW
CH = 80
NCHUNK = EPW // CH
NP = 10240
RPT = NP // NS
DEGW = 16

_sc_mesh = plsc.VectorSubcoreMesh(core_axis_name="c", subcore_axis_name="s")


@functools.partial(
    pl.kernel,
    out_type=jax.ShapeDtypeStruct((NC, NP, DEGW), jnp.float32),
    mesh=_sc_mesh,
    compiler_params=pltpu.CompilerParams(use_tc_tiling_on_sc=False),
    scratch_types=[
        pltpu.VMEM((CH,), jnp.int32),
        pltpu.VMEM((CH, DEGW), jnp.float32),
        pltpu.VMEM((RPT, DEGW), jnp.float32),
        pltpu.VMEM_SHARED((NP, DEGW), jnp.float32),
    ],
)
def _sc_degree(dst_hbm, out_hbm, dst_v, ones_v, z_v, acc):
    cid = lax.axis_index("c")
    sid = lax.axis_index("s")
    wid = sid * NC + cid

    def fill(i, _):
        ones_v[i] = jnp.ones((DEGW,), jnp.float32)
        return _

    lax.fori_loop(0, CH, fill, None)

    def zfill(i, _):
        z_v[i] = jnp.zeros((DEGW,), jnp.float32)
        return _

    lax.fori_loop(0, RPT, zfill, None)
    pltpu.sync_copy(z_v, acc.at[pl.ds(sid * RPT, RPT)])
    plsc.subcore_barrier()

    def chunk(k, _):
        base = wid * EPW + k * CH
        pltpu.sync_copy(dst_hbm.at[pl.ds(base, CH)], dst_v)
        pltpu.sync_copy(ones_v, acc.at[dst_v], add=True)
        return _

    lax.fori_loop(0, NCHUNK, chunk, None)
    plsc.subcore_barrier()
    pltpu.sync_copy(acc.at[pl.ds(sid * RPT, RPT)], z_v)
    pltpu.sync_copy(z_v, out_hbm.at[cid, pl.ds(sid * RPT, RPT)])


@functools.partial(
    pl.kernel,
    out_type=jax.ShapeDtypeStruct((NC, NP, D), jnp.float32),
    mesh=_sc_mesh,
    compiler_params=pltpu.CompilerParams(use_tc_tiling_on_sc=False),
    scratch_types=[
        pltpu.VMEM((CH,), jnp.int32),
        pltpu.VMEM((CH,), jnp.int32),
        pltpu.VMEM((CH, D), jnp.float32),
        pltpu.VMEM((RPT, D), jnp.float32),
        pltpu.SemaphoreType.DMA,
        pltpu.VMEM_SHARED((NP, D), jnp.float32),
    ],
)
def _sc_edge_pass(g_hbm, src_hbm, dst_hbm, out_hbm, src_v, dst_v, rows_v,
                  z_v, sem, acc):
    cid = lax.axis_index("c")
    sid = lax.axis_index("s")
    wid = sid * NC + cid

    def zfill(i, _):
        for j in range(D // 16):
            z_v[i, pl.ds(j * 16, 16)] = jnp.zeros((16,), jnp.float32)
        return _

    lax.fori_loop(0, RPT, zfill, None)
    pltpu.sync_copy(z_v, acc.at[pl.ds(sid * RPT, RPT)])
    plsc.subcore_barrier()

    def chunk(k, _):
        base = wid * EPW + k * CH
        pltpu.sync_copy(src_hbm.at[pl.ds(base, CH)], src_v)
        pltpu.sync_copy(dst_hbm.at[pl.ds(base, CH)], dst_v)
        pltpu.async_copy(g_hbm.at[src_v], rows_v, sem).wait()
        pltpu.sync_copy(rows_v, acc.at[dst_v], add=True)
        return _

    lax.fori_loop(0, NCHUNK, chunk, None)
    plsc.subcore_barrier()
    pltpu.sync_copy(acc.at[pl.ds(sid * RPT, RPT)], z_v)
    pltpu.sync_copy(z_v, out_hbm.at[cid, pl.ds(sid * RPT, RPT)])


_R = 2000


def _dinv_of(dp_ref):
    cnt = dp_ref[0, :, 0:1] + dp_ref[1, :, 0:1] + 1.0
    return lax.rsqrt(cnt)


def _tc_g1(x, w1, degp):
    def body(x_ref, w_ref, dp_ref, g_ref):
        h = jnp.dot(x_ref[...], w_ref[...], preferred_element_type=jnp.float32)
        g_ref[...] = _dinv_of(dp_ref) * h

    return pl.pallas_call(
        body,
        grid=(N // _R,),
        in_specs=[
            pl.BlockSpec((_R, DIN), lambda i: (i, 0)),
            pl.BlockSpec((DIN, D), lambda i: (0, 0)),
            pl.BlockSpec((NC, _R, DEGW), lambda i: (0, i, 0)),
        ],
        out_specs=pl.BlockSpec((_R, D), lambda i: (i, 0)),
        out_shape=jax.ShapeDtypeStruct((N, D), jnp.float32),
    )(x, w1, degp)


def _tc_mid(s, g, degp, w, b):
    def body(s_ref, g_ref, dp_ref, w_ref, b_ref, o_ref):
        dinv = _dinv_of(dp_ref)
        h = jnp.maximum(
            dinv * (s_ref[0] + s_ref[1] + g_ref[...]) + b_ref[...], 0.0)
        o_ref[...] = dinv * jnp.dot(h, w_ref[...],
                                    preferred_element_type=jnp.float32)

    return pl.pallas_call(
        body,
        grid=(N // _R,),
        in_specs=[
            pl.BlockSpec((NC, _R, D), lambda i: (0, i, 0)),
            pl.BlockSpec((_R, D), lambda i: (i, 0)),
            pl.BlockSpec((NC, _R, DEGW), lambda i: (0, i, 0)),
            pl.BlockSpec((D, D), lambda i: (0, 0)),
            pl.BlockSpec((1, D), lambda i: (0, 0)),
        ],
        out_specs=pl.BlockSpec((_R, D), lambda i: (i, 0)),
        out_shape=jax.ShapeDtypeStruct((N, D), jnp.float32),
    )(s, g, degp, w, b)


def _tc_prescale(s, g, degp, b):
    def body(s_ref, g_ref, dp_ref, b_ref, o_ref):
        dinv = _dinv_of(dp_ref)
        h = jnp.maximum(
            dinv * (s_ref[0] + s_ref[1] + g_ref[...]) + b_ref[...], 0.0)
        o_ref[...] = dinv * h

    return pl.pallas_call(
        body,
        grid=(N // _R,),
        in_specs=[
            pl.BlockSpec((NC, _R, D), lambda i: (0, i, 0)),
            pl.BlockSpec((_R, D), lambda i: (i, 0)),
            pl.BlockSpec((NC, _R, DEGW), lambda i: (0, i, 0)),
            pl.BlockSpec((1, D), lambda i: (0, 0)),
        ],
        out_specs=pl.BlockSpec((_R, D), lambda i: (i, 0)),
        out_shape=jax.ShapeDtypeStruct((N, D), jnp.float32),
    )(s, g, degp, b)


def _tc_out(s, q, degp, w2, b2):
    def body(s_ref, q_ref, dp_ref, w_ref, b_ref, o_ref):
        dinv = _dinv_of(dp_ref)
        a = dinv * (s_ref[0] + s_ref[1] + q_ref[...])
        h = jnp.dot(a, w_ref[...], preferred_element_type=jnp.float32)
        o_ref[...] = jnp.maximum(h + b_ref[...], 0.0)

    return pl.pallas_call(
        body,
        grid=(N // _R,),
        in_specs=[
            pl.BlockSpec((NC, _R, D), lambda i: (0, i, 0)),
            pl.BlockSpec((_R, D), lambda i: (i, 0)),
            pl.BlockSpec((NC, _R, DEGW), lambda i: (0, i, 0)),
            pl.BlockSpec((D, DOUT), lambda i: (0, 0)),
            pl.BlockSpec((1, DOUT), lambda i: (0, 0)),
        ],
        out_specs=pl.BlockSpec((_R, DOUT), lambda i: (i, 0)),
        out_shape=jax.ShapeDtypeStruct((N, DOUT), jnp.float32),
    )(s, q, degp, w2, b2)


def kernel(x, edge_index, W1, b1, W3, b3, W2, b2):
    src = edge_index[0]
    dst = edge_index[1]
    degp = _sc_degree(dst)
    g1 = _tc_g1(x, W1, degp)
    s1 = _sc_edge_pass(g1, src, dst)
    g2 = _tc_mid(s1, g1, degp, W3, b1.reshape(1, D))
    s2 = _sc_edge_pass(g2, src, dst)
    q = _tc_prescale(s2, g2, degp, b3.reshape(1, D))
    s3 = _sc_edge_pass(q, src, dst)
    return _tc_out(s3, q, degp, W2, b2.reshape(1, DOUT))


# speedup vs baseline: 38.0271x; 2.9413x over previous
import functools

import jax
import jax.numpy as jnp
from jax import lax
from jax.experimental import pallas as pl
from jax.experimental.pallas import tpu as pltpu
from jax.experimental.pallas import tpu_sc as plsc

N = 10000
E = 320000
D = 64
DIN = 128
DOUT = 128
NC, NS = 2, 16
NW = NC * NS
EPW = E // NW
CH = 100
NCHUNK = EPW // CH
NB = 5
NG = NCHUNK // NB
NP = 10240
RPT = NP // NS
NE16 = EPW // 16

_sc_mesh = plsc.VectorSubcoreMesh(core_axis_name="c", subcore_axis_name="s")


@functools.partial(
    pl.kernel,
    out_type=jax.ShapeDtypeStruct((NW, NP), jnp.float32),
    mesh=_sc_mesh,
    compiler_params=pltpu.CompilerParams(use_tc_tiling_on_sc=False,
                                         needs_layout_passes=False),
    scratch_types=[
        pltpu.VMEM((NE16, 16), jnp.int32),
        pltpu.VMEM((NP,), jnp.float32),
    ],
)
def _sc_degree(dst_hbm, out_hbm, dst_v, tacc):
    cid = lax.axis_index("c")
    sid = lax.axis_index("s")
    wid = sid * NC + cid
    pltpu.sync_copy(dst_hbm.at[wid], dst_v)

    def zfill(i, _):
        tacc[pl.ds(i * 16, 16)] = jnp.zeros((16,), jnp.float32)
        return _

    lax.fori_loop(0, NP // 16, zfill, None)

    def count(i, _):
        idx = dst_v[i]
        cnt, last = plsc.scan_count(idx)
        plsc.addupdate_scatter(tacc, [idx], cnt.astype(jnp.float32),
                               mask=last)
        return _

    lax.fori_loop(0, NE16, count, None)
    pltpu.sync_copy(tacc, out_hbm.at[wid])


@functools.partial(
    pl.kernel,
    out_type=jax.ShapeDtypeStruct((NC, NP, D), jnp.float32),
    mesh=_sc_mesh,
    compiler_params=pltpu.CompilerParams(use_tc_tiling_on_sc=False),
    scratch_types=[
        pltpu.VMEM((NCHUNK, CH), jnp.int32),
        pltpu.VMEM((NCHUNK, CH), jnp.int32),
        pltpu.VMEM((NB, CH, D), jnp.float32),
        pltpu.VMEM((80, D), jnp.float32),
        pltpu.SemaphoreType.DMA((NB,)),
        pltpu.SemaphoreType.DMA((NB,)),
        pltpu.VMEM_SHARED((NP, D), jnp.float32),
    ],
)
def _sc_edge_pass(g_hbm, src_hbm, dst_hbm, out_hbm, src_v, dst_v, rows_v,
                  z_v, gsem, ssem, acc):
    cid = lax.axis_index("c")
    sid = lax.axis_index("s")
    wid = sid * NC + cid
    pltpu.sync_copy(src_hbm.at[wid], src_v)
    pltpu.sync_copy(dst_hbm.at[wid], dst_v)

    def zfill(i, _):
        for j in range(D // 16):
            z_v[i, pl.ds(j * 16, 16)] = jnp.zeros((16,), jnp.float32)
        return _

    lax.fori_loop(0, 80, zfill, None)

    def zcopy(c, _):
        pltpu.sync_copy(z_v, acc.at[pl.ds(sid * RPT + c * 80, 80)])
        return _

    lax.fori_loop(0, RPT // 80, zcopy, None)
    plsc.subcore_barrier()

    for b in range(NB):
        pltpu.async_copy(g_hbm.at[src_v.at[b]], rows_v.at[b], gsem.at[b])

    def group(g, _):
        for b in range(NB):
            k = g * NB + b
            pltpu.make_async_copy(
                g_hbm.at[src_v.at[k]], rows_v.at[b], gsem.at[b]).wait()
            pltpu.async_copy(
                rows_v.at[b], acc.at[dst_v.at[k]], ssem.at[b], add=True)
            nk = k + NB

            @pl.when(nk < NCHUNK)
            def _start():
                pltpu.make_async_copy(
                    rows_v.at[b], acc.at[dst_v.at[k]], ssem.at[b]).wait()
                pltpu.async_copy(
                    g_hbm.at[src_v.at[nk]], rows_v.at[b], gsem.at[b])

        return _

    lax.fori_loop(0, NG, group, None)
    for b in range(NB):
        pltpu.make_async_copy(
            rows_v.at[b], acc.at[dst_v.at[0]], ssem.at[b]).wait()
    plsc.subcore_barrier()
    pltpu.sync_copy(acc.at[pl.ds(sid * RPT, RPT)],
                    out_hbm.at[cid, pl.ds(sid * RPT, RPT)])


_R = 2048


def _dinv_of(dp_ref):
    ones = jnp.ones((NW, 1), jnp.float32)
    cnt = lax.dot_general(dp_ref[...], ones, (((0,), (0,)), ((), ())),
                          preferred_element_type=jnp.float32)
    return lax.rsqrt(cnt + 1.0)


def _tc_g1(x, w1, degp):
    def body(x_ref, w_ref, dp_ref, g_ref):
        h = jnp.dot(x_ref[...], w_ref[...], preferred_element_type=jnp.float32)
        g_ref[...] = _dinv_of(dp_ref) * h

    return pl.pallas_call(
        body,
        grid=(NP // _R,),
        in_specs=[
            pl.BlockSpec((_R, DIN), lambda i: (i, 0)),
            pl.BlockSpec((DIN, D), lambda i: (0, 0)),
            pl.BlockSpec((NW, _R), lambda i: (0, i)),
        ],
        out_specs=pl.BlockSpec((_R, D), lambda i: (i, 0)),
        out_shape=jax.ShapeDtypeStruct((N, D), jnp.float32),
    )(x, w1, degp)


def _tc_mid(s, g, degp, w, b):
    def body(s_ref, g_ref, dp_ref, w_ref, b_ref, o_ref):
        dinv = _dinv_of(dp_ref)
        h = jnp.maximum(
            dinv * (s_ref[0] + s_ref[1] + g_ref[...]) + b_ref[...], 0.0)
        o_ref[...] = dinv * jnp.dot(h, w_ref[...],
                                    preferred_element_type=jnp.float32)

    return pl.pallas_call(
        body,
        grid=(NP // _R,),
        in_specs=[
            pl.BlockSpec((NC, _R, D), lambda i: (0, i, 0)),
            pl.BlockSpec((_R, D), lambda i: (i, 0)),
            pl.BlockSpec((NW, _R), lambda i: (0, i)),
            pl.BlockSpec((D, D), lambda i: (0, 0)),
            pl.BlockSpec((1, D), lambda i: (0, 0)),
        ],
        out_specs=pl.BlockSpec((_R, D), lambda i: (i, 0)),
        out_shape=jax.ShapeDtypeStruct((N, D), jnp.float32),
    )(s, g, degp, w, b)


def _tc_out(s, q, degp, w2, b2):
    def body(s_ref, q_ref, dp_ref, w_ref, b_ref, o_ref):
        dinv = _dinv_of(dp_ref)
        a = dinv * (s_ref[0] + s_ref[1] + q_ref[...])
        h = jnp.dot(a, w_ref[...], preferred_element_type=jnp.float32)
        o_ref[...] = jnp.maximum(h + b_ref[...], 0.0)

    return pl.pallas_call(
        body,
        grid=(NP // _R,),
        in_specs=[
            pl.BlockSpec((NC, _R, D), lambda i: (0, i, 0)),
            pl.BlockSpec((_R, D), lambda i: (i, 0)),
            pl.BlockSpec((NW, _R), lambda i: (0, i)),
            pl.BlockSpec((D, DOUT), lambda i: (0, 0)),
            pl.BlockSpec((1, DOUT), lambda i: (0, 0)),
        ],
        out_specs=pl.BlockSpec((_R, DOUT), lambda i: (i, 0)),
        out_shape=jax.ShapeDtypeStruct((N, DOUT), jnp.float32),
    )(s, q, degp, w2, b2)


def kernel(x, edge_index, W1, b1, W3, b3, W2, b2):
    src = edge_index[0].reshape(NW, NCHUNK, CH)
    dst = edge_index[1].reshape(NW, NCHUNK, CH)
    degp = _sc_degree(edge_index[1].reshape(NW, NE16, 16))
    g1 = _tc_g1(x, W1, degp)

    eye = jnp.eye(D, dtype=jnp.float32)
    wstack = jnp.stack([W3, eye, eye])
    bstack = jnp.stack([b1, b3, jnp.zeros_like(b3)]).reshape(3, 1, D)

    def body(i, carry):
        g, _, _ = carry
        s = _sc_edge_pass(g, src, dst)
        w = lax.dynamic_slice(wstack, (i, 0, 0), (1, D, D))[0]
        b = lax.dynamic_slice(bstack, (i, 0, 0), (1, 1, D))[0]
        gnew = _tc_mid(s, g, degp, w, b)
        return (gnew, g, s)

    s0 = jnp.zeros((NC, NP, D), jnp.float32)
    n_layers = lax.optimization_barrier(jnp.int32(3))
    _, q, s3 = lax.fori_loop(0, n_layers, body, (g1, g1, s0))
    return _tc_out(s3, q, degp, W2, b2.reshape(1, DOUT))


# speedup vs baseline: 43.5920x; 1.1463x over previous
import functools

import jax
import jax.numpy as jnp
from jax import lax
from jax.experimental import pallas as pl
from jax.experimental.pallas import tpu as pltpu
from jax.experimental.pallas import tpu_sc as plsc

N = 10000
E = 320000
D = 64
DIN = 128
DOUT = 128
NC, NS = 2, 16
NW = NC * NS
EPW = E // NW
CH = 100
NCHUNK = EPW // CH
NB = 5
NG = NCHUNK // NB
NP = 10240
RPT = NP // NS
NE16 = EPW // 16

_sc_mesh = plsc.VectorSubcoreMesh(core_axis_name="c", subcore_axis_name="s")


@functools.partial(
    pl.kernel,
    out_type=jax.ShapeDtypeStruct((NW, NP), jnp.float32),
    mesh=_sc_mesh,
    compiler_params=pltpu.CompilerParams(use_tc_tiling_on_sc=False,
                                         needs_layout_passes=False),
    scratch_types=[
        pltpu.VMEM((NE16, 16), jnp.int32),
        pltpu.VMEM((NP,), jnp.float32),
    ],
)
def _sc_degree(dst_hbm, out_hbm, dst_v, tacc):
    cid = lax.axis_index("c")
    sid = lax.axis_index("s")
    wid = sid * NC + cid
    pltpu.sync_copy(dst_hbm.at[wid], dst_v)

    def zfill(i, _):
        tacc[pl.ds(i * 16, 16)] = jnp.zeros((16,), jnp.float32)
        return _

    lax.fori_loop(0, NP // 16, zfill, None)

    def count(i, _):
        idx = dst_v[i]
        cnt, last = plsc.scan_count(idx)
        plsc.addupdate_scatter(tacc, [idx], cnt.astype(jnp.float32),
                               mask=last)
        return _

    lax.fori_loop(0, NE16, count, None)
    pltpu.sync_copy(tacc, out_hbm.at[wid])


@functools.partial(
    pl.kernel,
    out_type=jax.ShapeDtypeStruct((NC, NP, D), jnp.float32),
    mesh=_sc_mesh,
    compiler_params=pltpu.CompilerParams(use_tc_tiling_on_sc=False),
    scratch_types=[
        pltpu.VMEM((NCHUNK, CH), jnp.int32),
        pltpu.VMEM((NCHUNK, CH), jnp.int32),
        pltpu.VMEM((NB, CH, D), jnp.float32),
        pltpu.VMEM((80, D), jnp.float32),
        pltpu.SemaphoreType.DMA((NB,)),
        pltpu.SemaphoreType.DMA((NB,)),
        pltpu.VMEM_SHARED((NP, D), jnp.float32),
    ],
)
def _sc_edge_pass(g_hbm, src_hbm, dst_hbm, out_hbm, src_v, dst_v, rows_v,
                  z_v, gsem, ssem, acc):
    cid = lax.axis_index("c")
    sid = lax.axis_index("s")
    wid = sid * NC + cid
    pltpu.sync_copy(src_hbm.at[wid], src_v)
    pltpu.sync_copy(dst_hbm.at[wid], dst_v)

    def zfill(i, _):
        for j in range(D // 16):
            z_v[i, pl.ds(j * 16, 16)] = jnp.zeros((16,), jnp.float32)
        return _

    lax.fori_loop(0, 80, zfill, None)

    def zcopy(c, _):
        pltpu.sync_copy(z_v, acc.at[pl.ds(sid * RPT + c * 80, 80)])
        return _

    lax.fori_loop(0, RPT // 80, zcopy, None)
    plsc.subcore_barrier()

    for b in range(NB):
        pltpu.async_copy(g_hbm.at[src_v.at[b]], rows_v.at[b], gsem.at[b])

    def group(g, _):
        for b in range(NB):
            k = g * NB + b
            pltpu.make_async_copy(
                g_hbm.at[src_v.at[k]], rows_v.at[b], gsem.at[b]).wait()
            pltpu.async_copy(
                rows_v.at[b], acc.at[dst_v.at[k]], ssem.at[b], add=True)
            nk = k + NB

            @pl.when(nk < NCHUNK)
            def _start():
                pltpu.make_async_copy(
                    rows_v.at[b], acc.at[dst_v.at[k]], ssem.at[b]).wait()
                pltpu.async_copy(
                    g_hbm.at[src_v.at[nk]], rows_v.at[b], gsem.at[b])

        return _

    lax.fori_loop(0, NG, group, None)
    for b in range(NB):
        pltpu.make_async_copy(
            rows_v.at[b], acc.at[dst_v.at[0]], ssem.at[b]).wait()
    plsc.subcore_barrier()
    pltpu.sync_copy(acc.at[pl.ds(sid * RPT, RPT)],
                    out_hbm.at[cid, pl.ds(sid * RPT, RPT)])


_R = 2048


def _dinv_of(dp_ref):
    ones = jnp.ones((NW, 1), jnp.float32)
    cnt = lax.dot_general(dp_ref[...], ones, (((0,), (0,)), ((), ())),
                          preferred_element_type=jnp.float32)
    return lax.rsqrt(cnt + 1.0)


def _tc_g1(x, w1, degp):
    def body(x_ref, w_ref, dp_ref, g_ref):
        h = jnp.dot(x_ref[...], w_ref[...], preferred_element_type=jnp.float32)
        g_ref[...] = _dinv_of(dp_ref) * h

    return pl.pallas_call(
        body,
        grid=(NP // _R,),
        in_specs=[
            pl.BlockSpec((_R, DIN), lambda i: (i, 0)),
            pl.BlockSpec((DIN, D), lambda i: (0, 0)),
            pl.BlockSpec((NW, _R), lambda i: (0, i)),
        ],
        out_specs=pl.BlockSpec((_R, D), lambda i: (i, 0)),
        out_shape=jax.ShapeDtypeStruct((N, D), jnp.float32),
    )(x, w1, degp)


def _tc_mid(s, g, degp, w, b):
    def body(s_ref, g_ref, dp_ref, w_ref, b_ref, o_ref):
        dinv = _dinv_of(dp_ref)
        h = jnp.maximum(
            dinv * (s_ref[0] + s_ref[1] + g_ref[...]) + b_ref[...], 0.0)
        o_ref[...] = dinv * jnp.dot(h, w_ref[...],
                                    preferred_element_type=jnp.float32)

    return pl.pallas_call(
        body,
        grid=(NP // _R,),
        in_specs=[
            pl.BlockSpec((NC, _R, D), lambda i: (0, i, 0)),
            pl.BlockSpec((_R, D), lambda i: (i, 0)),
            pl.BlockSpec((NW, _R), lambda i: (0, i)),
            pl.BlockSpec((D, D), lambda i: (0, 0)),
            pl.BlockSpec((1, D), lambda i: (0, 0)),
        ],
        out_specs=pl.BlockSpec((_R, D), lambda i: (i, 0)),
        out_shape=jax.ShapeDtypeStruct((N, D), jnp.float32),
    )(s, g, degp, w, b)


def _tc_out(s, q, degp, w2, b2):
    def body(s_ref, q_ref, dp_ref, w_ref, b_ref, o_ref):
        dinv = _dinv_of(dp_ref)
        a = dinv * (s_ref[0] + s_ref[1] + q_ref[...])
        h = jnp.dot(a, w_ref[...], preferred_element_type=jnp.float32)
        o_ref[...] = jnp.maximum(h + b_ref[...], 0.0)

    return pl.pallas_call(
        body,
        grid=(NP // _R,),
        in_specs=[
            pl.BlockSpec((NC, _R, D), lambda i: (0, i, 0)),
            pl.BlockSpec((_R, D), lambda i: (i, 0)),
            pl.BlockSpec((NW, _R), lambda i: (0, i)),
            pl.BlockSpec((D, DOUT), lambda i: (0, 0)),
            pl.BlockSpec((1, DOUT), lambda i: (0, 0)),
        ],
        out_specs=pl.BlockSpec((_R, DOUT), lambda i: (i, 0)),
        out_shape=jax.ShapeDtypeStruct((N, DOUT), jnp.float32),
    )(s, q, degp, w2, b2)


def kernel(x, edge_index, W1, b1, W3, b3, W2, b2):
    src = edge_index[0].reshape(NW, NCHUNK, CH)
    dst = edge_index[1].reshape(NW, NCHUNK, CH)
    degp = _sc_degree(edge_index[1].reshape(NW, NE16, 16))
    g1 = _tc_g1(x, W1, degp)

    eye = jnp.eye(D, dtype=jnp.float32)
    wstack = jnp.stack([W3, eye, eye])
    bstack = jnp.stack([b1, b3, jnp.zeros_like(b3)]).reshape(3, 1, D)

    s1 = _sc_edge_pass(g1, src, dst)
    g2 = _tc_mid(s1, g1, degp, W3, b1.reshape(1, D))
    s2 = _sc_edge_pass(g2, src, dst)
    q = _tc_mid(s2, g2, degp, eye, b3.reshape(1, D))
    s3 = _sc_edge_pass(q, src, dst)
    return _tc_out(s3, q, degp, W2, b2.reshape(1, DOUT))


# speedup vs baseline: 44.2298x; 1.0146x over previous
import functools

import jax
import jax.numpy as jnp
from jax import lax
from jax.experimental import pallas as pl
from jax.experimental.pallas import tpu as pltpu
from jax.experimental.pallas import tpu_sc as plsc

N = 10000
E = 320000
D = 64
DIN = 128
DOUT = 128
NC, NS = 2, 16
NW = NC * NS
EPW = E // NW
CH = 100
NCHUNK = EPW // CH
NB = 10
NG = NCHUNK // NB
NP = 10240
RPT = NP // NS
NE16 = EPW // 16

_sc_mesh = plsc.VectorSubcoreMesh(core_axis_name="c", subcore_axis_name="s")


@functools.partial(
    pl.kernel,
    out_type=jax.ShapeDtypeStruct((NW, NP), jnp.float32),
    mesh=_sc_mesh,
    compiler_params=pltpu.CompilerParams(use_tc_tiling_on_sc=False,
                                         needs_layout_passes=False),
    scratch_types=[
        pltpu.VMEM((NE16, 16), jnp.int32),
        pltpu.VMEM((NP,), jnp.float32),
    ],
)
def _sc_degree(dst_hbm, out_hbm, dst_v, tacc):
    cid = lax.axis_index("c")
    sid = lax.axis_index("s")
    wid = sid * NC + cid
    pltpu.sync_copy(dst_hbm.at[wid], dst_v)

    def zfill(i, _):
        tacc[pl.ds(i * 16, 16)] = jnp.zeros((16,), jnp.float32)
        return _

    lax.fori_loop(0, NP // 16, zfill, None)

    def count(i, _):
        idx = dst_v[i]
        cnt, last = plsc.scan_count(idx)
        plsc.addupdate_scatter(tacc, [idx], cnt.astype(jnp.float32),
                               mask=last)
        return _

    lax.fori_loop(0, NE16, count, None)
    pltpu.sync_copy(tacc, out_hbm.at[wid])


@functools.partial(
    pl.kernel,
    out_type=jax.ShapeDtypeStruct((NC, NP, D), jnp.float32),
    mesh=_sc_mesh,
    compiler_params=pltpu.CompilerParams(use_tc_tiling_on_sc=False),
    scratch_types=[
        pltpu.VMEM((NCHUNK, CH), jnp.int32),
        pltpu.VMEM((NCHUNK, CH), jnp.int32),
        pltpu.VMEM((NB, CH, D), jnp.float32),
        pltpu.VMEM((80, D), jnp.float32),
        pltpu.SemaphoreType.DMA((NB,)),
        pltpu.SemaphoreType.DMA((NB,)),
        pltpu.SemaphoreType.DMA,
        pltpu.VMEM_SHARED((NP, D), jnp.float32),
    ],
)
def _sc_edge_pass(g_hbm, src_hbm, dst_hbm, out_hbm, src_v, dst_v, rows_v,
                  z_v, gsem, ssem, zsem, acc):
    cid = lax.axis_index("c")
    sid = lax.axis_index("s")
    wid = sid * NC + cid
    pltpu.sync_copy(src_hbm.at[wid], src_v)
    pltpu.sync_copy(dst_hbm.at[wid], dst_v)

    for b in range(NB):
        pltpu.async_copy(g_hbm.at[src_v.at[b]], rows_v.at[b], gsem.at[b])

    def zfill(i, _):
        for j in range(D // 16):
            z_v[i, pl.ds(j * 16, 16)] = jnp.zeros((16,), jnp.float32)
        return _

    lax.fori_loop(0, 80, zfill, None)
    for c in range(RPT // 80):
        pltpu.async_copy(z_v, acc.at[pl.ds(sid * RPT + c * 80, 80)], zsem)
    for c in range(RPT // 80):
        pltpu.make_async_copy(z_v, acc.at[pl.ds(sid * RPT, 80)], zsem).wait()
    plsc.subcore_barrier()

    def group(g, _):
        for b in range(NB):
            k = g * NB + b
            pltpu.make_async_copy(
                g_hbm.at[src_v.at[k]], rows_v.at[b], gsem.at[b]).wait()
            pltpu.async_copy(
                rows_v.at[b], acc.at[dst_v.at[k]], ssem.at[b], add=True)
            nk = k + NB

            @pl.when(nk < NCHUNK)
            def _start():
                pltpu.make_async_copy(
                    rows_v.at[b], acc.at[dst_v.at[k]], ssem.at[b]).wait()
                pltpu.async_copy(
                    g_hbm.at[src_v.at[nk]], rows_v.at[b], gsem.at[b])

        return _

    lax.fori_loop(0, NG, group, None)
    for b in range(NB):
        pltpu.make_async_copy(
            rows_v.at[b], acc.at[dst_v.at[0]], ssem.at[b]).wait()
    plsc.subcore_barrier()
    pltpu.sync_copy(acc.at[pl.ds(sid * RPT, RPT)],
                    out_hbm.at[cid, pl.ds(sid * RPT, RPT)])


_R = 2048


def _dinv_of(dp_ref):
    ones = jnp.ones((NW, 1), jnp.float32)
    cnt = lax.dot_general(dp_ref[...], ones, (((0,), (0,)), ((), ())),
                          preferred_element_type=jnp.float32)
    return lax.rsqrt(cnt + 1.0)


def _tc_matmul(x, w1):
    def body(x_ref, w_ref, u_ref):
        u_ref[...] = jnp.dot(x_ref[...], w_ref[...],
                             preferred_element_type=jnp.float32)

    return pl.pallas_call(
        body,
        grid=(NP // _R,),
        in_specs=[
            pl.BlockSpec((_R, DIN), lambda i: (i, 0)),
            pl.BlockSpec((DIN, D), lambda i: (0, 0)),
        ],
        out_specs=pl.BlockSpec((_R, D), lambda i: (i, 0)),
        out_shape=jax.ShapeDtypeStruct((N, D), jnp.float32),
    )(x, w1)


def _tc_scale(u, degp):
    def body(u_ref, dp_ref, g_ref):
        g_ref[...] = _dinv_of(dp_ref) * u_ref[...]

    return pl.pallas_call(
        body,
        grid=(NP // _R,),
        in_specs=[
            pl.BlockSpec((_R, D), lambda i: (i, 0)),
            pl.BlockSpec((NW, _R), lambda i: (0, i)),
        ],
        out_specs=pl.BlockSpec((_R, D), lambda i: (i, 0)),
        out_shape=jax.ShapeDtypeStruct((N, D), jnp.float32),
    )(u, degp)


def _tc_mid(s, g, degp, w, b):
    def body(s_ref, g_ref, dp_ref, w_ref, b_ref, o_ref):
        dinv = _dinv_of(dp_ref)
        h = jnp.maximum(
            dinv * (s_ref[0] + s_ref[1] + g_ref[...]) + b_ref[...], 0.0)
        o_ref[...] = dinv * jnp.dot(h, w_ref[...],
                                    preferred_element_type=jnp.float32)

    return pl.pallas_call(
        body,
        grid=(NP // _R,),
        in_specs=[
            pl.BlockSpec((NC, _R, D), lambda i: (0, i, 0)),
            pl.BlockSpec((_R, D), lambda i: (i, 0)),
            pl.BlockSpec((NW, _R), lambda i: (0, i)),
            pl.BlockSpec((D, D), lambda i: (0, 0)),
            pl.BlockSpec((1, D), lambda i: (0, 0)),
        ],
        out_specs=pl.BlockSpec((_R, D), lambda i: (i, 0)),
        out_shape=jax.ShapeDtypeStruct((N, D), jnp.float32),
    )(s, g, degp, w, b)


def _tc_out(s, q, degp, w2, b2):
    def body(s_ref, q_ref, dp_ref, w_ref, b_ref, o_ref):
        dinv = _dinv_of(dp_ref)
        a = dinv * (s_ref[0] + s_ref[1] + q_ref[...])
        h = jnp.dot(a, w_ref[...], preferred_element_type=jnp.float32)
        o_ref[...] = jnp.maximum(h + b_ref[...], 0.0)

    return pl.pallas_call(
        body,
        grid=(NP // _R,),
        in_specs=[
            pl.BlockSpec((NC, _R, D), lambda i: (0, i, 0)),
            pl.BlockSpec((_R, D), lambda i: (i, 0)),
            pl.BlockSpec((NW, _R), lambda i: (0, i)),
            pl.BlockSpec((D, DOUT), lambda i: (0, 0)),
            pl.BlockSpec((1, DOUT), lambda i: (0, 0)),
        ],
        out_specs=pl.BlockSpec((_R, DOUT), lambda i: (i, 0)),
        out_shape=jax.ShapeDtypeStruct((N, DOUT), jnp.float32),
    )(s, q, degp, w2, b2)


def kernel(x, edge_index, W1, b1, W3, b3, W2, b2):
    src = edge_index[0].reshape(NW, NCHUNK, CH)
    dst = edge_index[1].reshape(NW, NCHUNK, CH)
    u1 = _tc_matmul(x, W1)
    degp = _sc_degree(edge_index[1].reshape(NW, NE16, 16))
    g1 = _tc_scale(u1, degp)

    eye = jnp.eye(D, dtype=jnp.float32)
    wstack = jnp.stack([W3, eye, eye])
    bstack = jnp.stack([b1, b3, jnp.zeros_like(b3)]).reshape(3, 1, D)

    s1 = _sc_edge_pass(g1, src, dst)
    g2 = _tc_mid(s1, g1, degp, W3, b1.reshape(1, D))
    s2 = _sc_edge_pass(g2, src, dst)
    q = _tc_mid(s2, g2, degp, eye, b3.reshape(1, D))
    s3 = _sc_edge_pass(q, src, dst)
    return _tc_out(s3, q, degp, W2, b2.reshape(1, DOUT))
